# baseline (speedup 1.0000x reference)
import jax
import jax.numpy as jnp
from jax import lax
from jax.experimental import pallas as pl
from jax.experimental.pallas import tpu as pltpu


def _body(idx_ref, table, tag, w1, b1, w2, b2, out_ref,
          row_v, tag_v, w1_v, b1_v, w2_v, b2_v, sems):
    i = idx_ref[0]
    copies = (
        pltpu.make_async_copy(table.at[pl.ds(i, 1)], row_v, sems.at[0]),
        pltpu.make_async_copy(tag, tag_v, sems.at[1]),
        pltpu.make_async_copy(w1, w1_v, sems.at[2]),
        pltpu.make_async_copy(b1, b1_v, sems.at[3]),
        pltpu.make_async_copy(w2, w2_v, sems.at[4]),
        pltpu.make_async_copy(b2, b2_v, sems.at[5]),
    )
    for c in copies:
        c.start()
    for c in copies:
        c.wait()

    cat = jnp.concatenate([row_v[...], tag_v[...]], axis=1)
    z1 = lax.dot_general(
        cat, w1_v[...], (((1,), (1,)), ((), ())),
        preferred_element_type=jnp.float32,
    ) + b1_v[...]
    a1 = jnp.maximum(z1, 0.0)
    z2 = lax.dot_general(
        a1, w2_v[...], (((1,), (1,)), ((), ())),
        preferred_element_type=jnp.float32,
    ) + b2_v[...]
    m = jnp.max(z2, axis=1, keepdims=True)
    s = jnp.sum(jnp.exp(z2 - m), axis=1, keepdims=True)
    out_ref[...] = z2 - m - jnp.log(s)


@jax.jit
def kernel(word_embed_idx, pre_tag_embed, table, W1, b1, W2, b2):
    idx = word_embed_idx.astype(jnp.int32)
    grid_spec = pltpu.PrefetchScalarGridSpec(
        num_scalar_prefetch=1,
        grid=(1,),
        in_specs=[pl.BlockSpec(memory_space=pl.ANY)] * 6,
        out_specs=pl.BlockSpec((1, 64), lambda i, idx_ref: (0, 0)),
        scratch_shapes=[
            pltpu.VMEM((1, 128), jnp.float32),
            pltpu.VMEM((1, 64), jnp.float32),
            pltpu.VMEM((256, 192), jnp.float32),
            pltpu.VMEM((1, 256), jnp.float32),
            pltpu.VMEM((64, 256), jnp.float32),
            pltpu.VMEM((1, 64), jnp.float32),
            pltpu.SemaphoreType.DMA((6,)),
        ],
    )
    return pl.pallas_call(
        _body,
        grid_spec=grid_spec,
        out_shape=jax.ShapeDtypeStruct((1, 64), jnp.float32),
    )(idx, table, pre_tag_embed, W1, b1.reshape(1, -1), W2, b2.reshape(1, -1))
